# baseline (speedup 1.0000x reference)
import functools
import math

import jax
import jax.numpy as jnp
from jax import lax
from jax.experimental import pallas as pl
from jax.experimental.pallas import tpu as pltpu
from jax.experimental.pallas import tpu_sc as plsc

D_MODEL = 64
SCALE = math.sqrt(D_MODEL)
NUM_CORES = 2
NUM_SUBCORES = 16
NUM_WORKERS = NUM_CORES * NUM_SUBCORES
LANES = 16
VOCAB = 1000000
KROWS = VOCAB // 2
KBLK = 128
KBLK_PER_W = 123
K0_MAX = 499904
WIDE_ROWS = K0_MAX + KBLK
NBUF = 3


def _mesh():
    return plsc.VectorSubcoreMesh(core_axis_name="c", subcore_axis_name="s")


def _widen(table_t):
    @functools.partial(
        pl.kernel,
        mesh=_mesh(),
        out_type=jax.ShapeDtypeStruct((WIDE_ROWS, 128), jnp.float32),
        scratch_types=(
            [pltpu.VMEM((D_MODEL, 2 * KBLK), jnp.float32) for _ in range(NBUF)]
            + [pltpu.VMEM((KBLK, 128), jnp.float32) for _ in range(NBUF)]
            + [pltpu.SemaphoreType.DMA for _ in range(2 * NBUF)]
        ),
        compiler_params=pltpu.CompilerParams(needs_layout_passes=False),
    )
    def k1(tt_hbm, wide_hbm, *refs):
        srcs = refs[0:NBUF]
        dsts = refs[NBUF:2 * NBUF]
        gsems = refs[2 * NBUF:3 * NBUF]
        wsems = refs[3 * NBUF:4 * NBUF]
        wid = lax.axis_index("s") * NUM_CORES + lax.axis_index("c")
        iota = lax.iota(jnp.int32, LANES)
        rbs = [iota + di * LANES for di in range(D_MODEL // LANES)]

        def k0_of(i):
            return jnp.minimum((i * NUM_WORKERS + wid) * KBLK, K0_MAX)

        def start_load(i, b):
            off = pl.multiple_of(2 * k0_of(i), KBLK)
            pltpu.async_copy(
                tt_hbm.at[:, pl.ds(off, 2 * KBLK)], srcs[b],
                gsems[b])

        for b in range(NBUF):
            start_load(b, b)

        def transpose_block(b):
            def diag_body(c, cr):
                pc = (iota + c) & (LANES - 1)
                for vj in range(2 * KBLK // LANES):
                    vl = pc + vj * LANES
                    row = lax.shift_right_logical(vl, 1)
                    cb = lax.shift_left(vl & 1, 6)
                    for di in range(D_MODEL // LANES):
                        v = plsc.load_gather(srcs[b], [rbs[di], vl])
                        plsc.store_scatter(dsts[b], [row, cb + rbs[di]], v)
                return cr

            lax.fori_loop(0, LANES, diag_body, 0)

        def blk_body(r, carry):
            for b in range(NBUF):
                i = r * NBUF + b
                pltpu.make_async_copy(
                    tt_hbm.at[:, pl.ds(0, 2 * KBLK)], srcs[b],
                    gsems[b]).wait()

                @pl.when(i >= NBUF)
                def _():
                    pltpu.make_async_copy(
                        dsts[b], wide_hbm.at[pl.ds(0, KBLK), :],
                        wsems[b]).wait()

                transpose_block(b)

                @pl.when(i + NBUF < KBLK_PER_W)
                def _():
                    start_load(i + NBUF, b)

                pltpu.async_copy(
                    dsts[b],
                    wide_hbm.at[pl.ds(pl.multiple_of(k0_of(i), 8), KBLK), :],
                    wsems[b])
            return carry

        lax.fori_loop(0, KBLK_PER_W // NBUF, blk_body, 0)

        for bb in range(NBUF):
            pltpu.make_async_copy(
                dsts[bb], wide_hbm.at[pl.ds(0, KBLK), :], wsems[bb]).wait()

    return k1(table_t)


def _gather(x_t, wide):
    n_pos, n_batch = x_t.shape
    rblk = n_batch // NUM_WORKERS

    @functools.partial(
        pl.kernel,
        mesh=_mesh(),
        out_type=jax.ShapeDtypeStruct((n_pos, D_MODEL, n_batch), jnp.float32),
        scratch_types=(
            [pltpu.VMEM((n_pos, rblk), jnp.int32),
             pltpu.VMEM((n_pos, rblk), jnp.int32)]
            + [pltpu.VMEM((rblk, 128), jnp.float32) for _ in range(NBUF)]
            + [pltpu.VMEM((D_MODEL, rblk), jnp.float32) for _ in range(NBUF)]
            + [pltpu.SemaphoreType.DMA for _ in range(2 * NBUF)]
        ),
        compiler_params=pltpu.CompilerParams(needs_layout_passes=False),
    )
    def k2(xt_hbm, wide_hbm, out_hbm, idxv, idxh, *refs):
        ins = refs[0:NBUF]
        obs = refs[NBUF:2 * NBUF]
        gsems = refs[2 * NBUF:3 * NBUF]
        wsems = refs[3 * NBUF:4 * NBUF]
        wid = lax.axis_index("s") * NUM_CORES + lax.axis_index("c")
        r0 = wid * rblk
        iota = lax.iota(jnp.int32, LANES)
        rbs = [iota + ri * LANES for ri in range(rblk // LANES)]

        pltpu.sync_copy(xt_hbm.at[:, pl.ds(r0, rblk)], idxv)

        def half_body(t, c):
            for j in range(rblk // LANES):
                sl = pl.ds(j * LANES, LANES)
                idxh[t, sl] = lax.shift_right_logical(idxv[t, sl], 1)
            return c

        lax.fori_loop(0, n_pos, half_body, 0)

        def start_gather(t, b):
            pltpu.async_copy(wide_hbm.at[idxh.at[t]], ins[b], gsems[b])

        for b in range(NBUF):
            start_gather(b, b)

        def transpose_block(t, b):
            pars = [lax.shift_left(idxv[t, pl.ds(ri * LANES, LANES)] & 1, 6)
                    for ri in range(rblk // LANES)]

            def diag_body(c, cr):
                pc = (iota + c) & (LANES - 1)
                for dj in range(D_MODEL // LANES):
                    dcol = pc + dj * LANES
                    for ri in range(rblk // LANES):
                        v = plsc.load_gather(
                            ins[b], [rbs[ri], dcol + pars[ri]]) * SCALE
                        plsc.store_scatter(obs[b], [dcol, rbs[ri]], v)
                return cr

            lax.fori_loop(0, LANES, diag_body, 0)

        def t_body(r, carry):
            for b in range(NBUF):
                t = r * NBUF + b

                @pl.when(t < n_pos)
                def _():
                    pltpu.make_async_copy(
                        wide_hbm.at[idxh.at[0]], ins[b], gsems[b]).wait()

                    @pl.when(t >= NBUF)
                    def _():
                        pltpu.make_async_copy(
                            obs[b], out_hbm.at[0, :, pl.ds(0, rblk)],
                            wsems[b]).wait()

                    transpose_block(t, b)

                    @pl.when(t + NBUF < n_pos)
                    def _():
                        start_gather(t + NBUF, b)

                    pltpu.async_copy(
                        obs[b], out_hbm.at[t, :, pl.ds(r0, rblk)], wsems[b])
            return carry

        lax.fori_loop(0, (n_pos + NBUF - 1) // NBUF, t_body, 0)

        for bb in range(NBUF):
            pltpu.make_async_copy(
                obs[bb], out_hbm.at[0, :, pl.ds(0, rblk)], wsems[bb]).wait()

    return k2(x_t, wide)


def kernel(x, table):
    x_t = x.astype(jnp.int32).T
    table_t = table.T
    wide = _widen(table_t)
    out_t = _gather(x_t, wide)
    return jnp.transpose(out_t, (2, 0, 1))


# speedup vs baseline: 1.3700x; 1.3700x over previous
import functools
import math

import jax
import jax.numpy as jnp
from jax import lax
from jax.experimental import pallas as pl
from jax.experimental.pallas import tpu as pltpu
from jax.experimental.pallas import tpu_sc as plsc

D_MODEL = 64
SCALE = math.sqrt(D_MODEL)
NUM_CORES = 2
NUM_SUBCORES = 16
NUM_WORKERS = NUM_CORES * NUM_SUBCORES
LANES = 16
VOCAB = 1000000
KROWS = VOCAB // 2
KBLK = 128
KBLK_PER_W = 123
K0_MAX = 499904
WIDE_ROWS = K0_MAX + KBLK
NBUF = 3


def _mesh():
    return plsc.VectorSubcoreMesh(core_axis_name="c", subcore_axis_name="s")


def _widen(table_t):
    @functools.partial(
        pl.kernel,
        mesh=_mesh(),
        out_type=jax.ShapeDtypeStruct((WIDE_ROWS, 128), jnp.float32),
        scratch_types=(
            [pltpu.VMEM((D_MODEL, 2 * KBLK), jnp.float32) for _ in range(NBUF)]
            + [pltpu.VMEM((KBLK, 128), jnp.float32) for _ in range(NBUF)]
            + [pltpu.SemaphoreType.DMA for _ in range(2 * NBUF)]
        ),
        compiler_params=pltpu.CompilerParams(needs_layout_passes=False),
    )
    def k1(tt_hbm, wide_hbm, *refs):
        srcs = refs[0:NBUF]
        dsts = refs[NBUF:2 * NBUF]
        gsems = refs[2 * NBUF:3 * NBUF]
        wsems = refs[3 * NBUF:4 * NBUF]
        wid = lax.axis_index("s") * NUM_CORES + lax.axis_index("c")
        iota = lax.iota(jnp.int32, LANES)
        rbs = [iota + di * LANES for di in range(D_MODEL // LANES)]

        def k0_of(i):
            return jnp.minimum((i * NUM_WORKERS + wid) * KBLK, K0_MAX)

        def start_load(i, b):
            off = pl.multiple_of(2 * k0_of(i), KBLK)
            pltpu.async_copy(
                tt_hbm.at[:, pl.ds(off, 2 * KBLK)], srcs[b],
                gsems[b])

        for b in range(NBUF):
            start_load(b, b)

        def transpose_block(b):
            @plsc.parallel_loop(0, LANES, unroll=2)
            def diag_body(c):
                pc = (iota + c) & (LANES - 1)
                for vj in range(2 * KBLK // LANES):
                    vl = pc + vj * LANES
                    row = lax.shift_right_logical(vl, 1)
                    cb = lax.shift_left(vl & 1, 6)
                    for di in range(D_MODEL // LANES):
                        v = plsc.load_gather(srcs[b], [rbs[di], vl])
                        plsc.store_scatter(dsts[b], [row, cb + rbs[di]], v)

        def blk_body(r, carry):
            for b in range(NBUF):
                i = r * NBUF + b
                pltpu.make_async_copy(
                    tt_hbm.at[:, pl.ds(0, 2 * KBLK)], srcs[b],
                    gsems[b]).wait()

                @pl.when(i >= NBUF)
                def _():
                    pltpu.make_async_copy(
                        dsts[b], wide_hbm.at[pl.ds(0, KBLK), :],
                        wsems[b]).wait()

                transpose_block(b)

                @pl.when(i + NBUF < KBLK_PER_W)
                def _():
                    start_load(i + NBUF, b)

                pltpu.async_copy(
                    dsts[b],
                    wide_hbm.at[pl.ds(pl.multiple_of(k0_of(i), 8), KBLK), :],
                    wsems[b])
            return carry

        lax.fori_loop(0, KBLK_PER_W // NBUF, blk_body, 0)

        for bb in range(NBUF):
            pltpu.make_async_copy(
                dsts[bb], wide_hbm.at[pl.ds(0, KBLK), :], wsems[bb]).wait()

    return k1(table_t)


def _gather(x_t, wide):
    n_pos, n_batch = x_t.shape
    rblk = n_batch // NUM_WORKERS

    @functools.partial(
        pl.kernel,
        mesh=_mesh(),
        out_type=jax.ShapeDtypeStruct((n_pos, D_MODEL, n_batch), jnp.float32),
        scratch_types=(
            [pltpu.VMEM((n_pos, rblk), jnp.int32),
             pltpu.VMEM((n_pos, rblk), jnp.int32)]
            + [pltpu.VMEM((rblk, 128), jnp.float32) for _ in range(NBUF)]
            + [pltpu.VMEM((D_MODEL, rblk), jnp.float32) for _ in range(NBUF)]
            + [pltpu.SemaphoreType.DMA for _ in range(2 * NBUF)]
        ),
        compiler_params=pltpu.CompilerParams(needs_layout_passes=False),
    )
    def k2(xt_hbm, wide_hbm, out_hbm, idxv, idxh, *refs):
        ins = refs[0:NBUF]
        obs = refs[NBUF:2 * NBUF]
        gsems = refs[2 * NBUF:3 * NBUF]
        wsems = refs[3 * NBUF:4 * NBUF]
        wid = lax.axis_index("s") * NUM_CORES + lax.axis_index("c")
        r0 = wid * rblk
        iota = lax.iota(jnp.int32, LANES)
        rbs = [iota + ri * LANES for ri in range(rblk // LANES)]

        pltpu.sync_copy(xt_hbm.at[:, pl.ds(r0, rblk)], idxv)

        def half_body(t, c):
            for j in range(rblk // LANES):
                sl = pl.ds(j * LANES, LANES)
                idxh[t, sl] = lax.shift_right_logical(idxv[t, sl], 1)
            return c

        lax.fori_loop(0, n_pos, half_body, 0)

        def start_gather(t, b):
            pltpu.async_copy(wide_hbm.at[idxh.at[t]], ins[b], gsems[b])

        for b in range(NBUF):
            start_gather(b, b)

        def transpose_block(t, b):
            pars = [lax.shift_left(idxv[t, pl.ds(ri * LANES, LANES)] & 1, 6)
                    for ri in range(rblk // LANES)]

            @plsc.parallel_loop(0, LANES, unroll=2)
            def diag_body(c):
                pc = (iota + c) & (LANES - 1)
                for dj in range(D_MODEL // LANES):
                    dcol = pc + dj * LANES
                    for ri in range(rblk // LANES):
                        v = plsc.load_gather(
                            ins[b], [rbs[ri], dcol + pars[ri]]) * SCALE
                        plsc.store_scatter(obs[b], [dcol, rbs[ri]], v)

        def t_body(r, carry):
            for b in range(NBUF):
                t = r * NBUF + b

                @pl.when(t < n_pos)
                def _():
                    pltpu.make_async_copy(
                        wide_hbm.at[idxh.at[0]], ins[b], gsems[b]).wait()

                    @pl.when(t >= NBUF)
                    def _():
                        pltpu.make_async_copy(
                            obs[b], out_hbm.at[0, :, pl.ds(0, rblk)],
                            wsems[b]).wait()

                    transpose_block(t, b)

                    @pl.when(t + NBUF < n_pos)
                    def _():
                        start_gather(t + NBUF, b)

                    pltpu.async_copy(
                        obs[b], out_hbm.at[t, :, pl.ds(r0, rblk)], wsems[b])
            return carry

        lax.fori_loop(0, (n_pos + NBUF - 1) // NBUF, t_body, 0)

        for bb in range(NBUF):
            pltpu.make_async_copy(
                obs[bb], out_hbm.at[0, :, pl.ds(0, rblk)], wsems[bb]).wait()

    return k2(x_t, wide)


def kernel(x, table):
    x_t = x.astype(jnp.int32).T
    table_t = table.T
    wide = _widen(table_t)
    out_t = _gather(x_t, wide)
    return jnp.transpose(out_t, (2, 0, 1))


# speedup vs baseline: 2.1525x; 1.5711x over previous
import functools
import math

import jax
import jax.numpy as jnp
from jax import lax
from jax.experimental import pallas as pl
from jax.experimental.pallas import tpu as pltpu
from jax.experimental.pallas import tpu_sc as plsc

D_MODEL = 64
SCALE = math.sqrt(D_MODEL)
NUM_CORES = 2
NUM_SUBCORES = 16
NUM_WORKERS = NUM_CORES * NUM_SUBCORES
LANES = 16
VOCAB = 1000000
KROWS = VOCAB // 2
KBLK = 128
KBLK_PER_W = 123
K0_MAX = 499904
WIDE_ROWS = K0_MAX + KBLK
NBUF = 3


def _mesh():
    return plsc.VectorSubcoreMesh(core_axis_name="c", subcore_axis_name="s")


def _widen(table_t):
    @functools.partial(
        pl.kernel,
        mesh=_mesh(),
        out_type=jax.ShapeDtypeStruct((WIDE_ROWS, 128), jnp.float32),
        scratch_types=(
            [pltpu.VMEM((D_MODEL, 2 * KBLK), jnp.float32) for _ in range(NBUF)]
            + [pltpu.VMEM((KBLK, 128), jnp.float32) for _ in range(NBUF)]
            + [pltpu.SemaphoreType.DMA for _ in range(2 * NBUF)]
        ),
        compiler_params=pltpu.CompilerParams(needs_layout_passes=False),
    )
    def k1(tt_hbm, wide_hbm, *refs):
        srcs = refs[0:NBUF]
        dsts = refs[NBUF:2 * NBUF]
        gsems = refs[2 * NBUF:3 * NBUF]
        wsems = refs[3 * NBUF:4 * NBUF]
        wid = lax.axis_index("s") * NUM_CORES + lax.axis_index("c")
        iota = lax.iota(jnp.int32, LANES)
        rbs = [iota + di * LANES for di in range(D_MODEL // LANES)]

        def k0_of(i):
            return jnp.minimum((i * NUM_WORKERS + wid) * KBLK, K0_MAX)

        def start_load(i, b):
            off = pl.multiple_of(2 * k0_of(i), KBLK)
            pltpu.async_copy(
                tt_hbm.at[:, pl.ds(off, 2 * KBLK)], srcs[b],
                gsems[b])

        for b in range(NBUF):
            start_load(b, b)

        def transpose_block(b):
            @plsc.parallel_loop(0, LANES, unroll=4)
            def diag_body(c):
                pc = (iota + c) & (LANES - 1)
                pch = lax.shift_right_logical(pc, 1)
                cb = lax.shift_left(pc & 1, 6)
                rbc = [rbs[di] + cb for di in range(D_MODEL // LANES)]
                for vj in range(2 * KBLK // LANES):
                    vl = pc + vj * LANES
                    row = pch + vj * (LANES // 2)
                    for di in range(D_MODEL // LANES):
                        v = plsc.load_gather(srcs[b], [rbs[di], vl])
                        plsc.store_scatter(dsts[b], [row, rbc[di]], v)

        def blk_body(r, carry):
            for b in range(NBUF):
                i = r * NBUF + b
                pltpu.make_async_copy(
                    tt_hbm.at[:, pl.ds(0, 2 * KBLK)], srcs[b],
                    gsems[b]).wait()

                @pl.when(i >= NBUF)
                def _():
                    pltpu.make_async_copy(
                        dsts[b], wide_hbm.at[pl.ds(0, KBLK), :],
                        wsems[b]).wait()

                transpose_block(b)

                @pl.when(i + NBUF < KBLK_PER_W)
                def _():
                    start_load(i + NBUF, b)

                pltpu.async_copy(
                    dsts[b],
                    wide_hbm.at[pl.ds(pl.multiple_of(k0_of(i), 8), KBLK), :],
                    wsems[b])
            return carry

        lax.fori_loop(0, KBLK_PER_W // NBUF, blk_body, 0)

        for bb in range(NBUF):
            pltpu.make_async_copy(
                dsts[bb], wide_hbm.at[pl.ds(0, KBLK), :], wsems[bb]).wait()

    return k1(table_t)


def _gather(x_t, wide):
    n_pos, n_batch = x_t.shape
    rblk = n_batch // NUM_WORKERS

    @functools.partial(
        pl.kernel,
        mesh=_mesh(),
        out_type=jax.ShapeDtypeStruct((n_pos, D_MODEL, n_batch), jnp.float32),
        scratch_types=(
            [pltpu.VMEM((n_pos, rblk), jnp.int32),
             pltpu.VMEM((n_pos, rblk), jnp.int32)]
            + [pltpu.VMEM((rblk, 128), jnp.float32) for _ in range(NBUF)]
            + [pltpu.VMEM((D_MODEL, rblk), jnp.float32) for _ in range(NBUF)]
            + [pltpu.SemaphoreType.DMA for _ in range(2 * NBUF)]
        ),
        compiler_params=pltpu.CompilerParams(needs_layout_passes=False),
    )
    def k2(xt_hbm, wide_hbm, out_hbm, idxv, idxh, *refs):
        ins = refs[0:NBUF]
        obs = refs[NBUF:2 * NBUF]
        gsems = refs[2 * NBUF:3 * NBUF]
        wsems = refs[3 * NBUF:4 * NBUF]
        wid = lax.axis_index("s") * NUM_CORES + lax.axis_index("c")
        r0 = wid * rblk
        iota = lax.iota(jnp.int32, LANES)
        rbs = [iota + ri * LANES for ri in range(rblk // LANES)]

        pltpu.sync_copy(xt_hbm.at[:, pl.ds(r0, rblk)], idxv)

        def half_body(t, c):
            for j in range(rblk // LANES):
                sl = pl.ds(j * LANES, LANES)
                idxh[t, sl] = lax.shift_right_logical(idxv[t, sl], 1)
            return c

        lax.fori_loop(0, n_pos, half_body, 0)

        def start_gather(t, b):
            pltpu.async_copy(wide_hbm.at[idxh.at[t]], ins[b], gsems[b])

        for b in range(NBUF):
            start_gather(b, b)

        def transpose_block(t, b):
            pars = [lax.shift_left(idxv[t, pl.ds(ri * LANES, LANES)] & 1, 6)
                    for ri in range(rblk // LANES)]

            @plsc.parallel_loop(0, LANES, unroll=2)
            def diag_body(c):
                pc = (iota + c) & (LANES - 1)
                for dj in range(D_MODEL // LANES):
                    dcol = pc + dj * LANES
                    for ri in range(rblk // LANES):
                        v = plsc.load_gather(
                            ins[b], [rbs[ri], dcol + pars[ri]]) * SCALE
                        plsc.store_scatter(obs[b], [dcol, rbs[ri]], v)

        def t_body(r, carry):
            for b in range(NBUF):
                t = r * NBUF + b

                @pl.when(t < n_pos)
                def _():
                    pltpu.make_async_copy(
                        wide_hbm.at[idxh.at[0]], ins[b], gsems[b]).wait()

                    @pl.when(t >= NBUF)
                    def _():
                        pltpu.make_async_copy(
                            obs[b], out_hbm.at[0, :, pl.ds(0, rblk)],
                            wsems[b]).wait()

                    transpose_block(t, b)

                    @pl.when(t + NBUF < n_pos)
                    def _():
                        start_gather(t + NBUF, b)

                    pltpu.async_copy(
                        obs[b], out_hbm.at[t, :, pl.ds(r0, rblk)], wsems[b])
            return carry

        lax.fori_loop(0, (n_pos + NBUF - 1) // NBUF, t_body, 0)

        for bb in range(NBUF):
            pltpu.make_async_copy(
                obs[bb], out_hbm.at[0, :, pl.ds(0, rblk)], wsems[bb]).wait()

    return k2(x_t, wide)


def kernel(x, table):
    x_t = x.astype(jnp.int32).T
    table_t = table.T
    wide = _widen(table_t)
    out_t = _gather(x_t, wide)
    return jnp.transpose(out_t, (2, 0, 1))
